# baseline (speedup 1.0000x reference)
import functools

import jax
import jax.numpy as jnp
from jax import lax
from jax.experimental import pallas as pl
from jax.experimental.pallas import tpu as pltpu
from jax.experimental.pallas import tpu_sc as plsc

_DIM = 32
_VPAD = 128
_C = 128
_K = 8
_NC = 2
_NS = 16
_NW = _NC * _NS
_ROWS_PER_W = _VPAD // _NW

_SC_PARAMS = pltpu.CompilerParams(
    needs_layout_passes=False, use_tc_tiling_on_sc=False)
_MESH = plsc.VectorSubcoreMesh(core_axis_name="c", subcore_axis_name="s")


@functools.partial(
    pl.kernel,
    mesh=_MESH,
    compiler_params=_SC_PARAMS,
    out_type=jax.ShapeDtypeStruct((_VPAD, _DIM), jnp.float32),
    scratch_types=[
        pltpu.VMEM((128,), jnp.float32),
        pltpu.VMEM((_DIM * _DIM // 128, 128), jnp.float32),
        pltpu.VMEM((128,), jnp.float32),
        pltpu.VMEM((_ROWS_PER_W, _DIM), jnp.float32),
    ],
)
def _build_table(e_hbm, w_hbm, b_hbm, t_hbm, e_vs, w_v, b_v, t_v):
    wid = lax.axis_index("s") * _NC + lax.axis_index("c")
    pltpu.sync_copy(e_hbm.at[wid], e_vs)
    pltpu.sync_copy(w_hbm, w_v)
    pltpu.sync_copy(b_hbm, b_v)
    b0 = b_v[pl.ds(0, 16)]
    b1 = b_v[pl.ds(16, 16)]
    for l in range(_ROWS_PER_W):
        acc0, acc1 = b0, b1
        for k in range(_DIM):
            ek = plsc.load_gather(
                e_vs, [jnp.full((16,), l * _DIM + k, dtype=jnp.int32)])
            wf = k * _DIM
            w0 = w_v[wf // 128, pl.ds(wf % 128, 16)]
            w1 = w_v[wf // 128, pl.ds(wf % 128 + 16, 16)]
            acc0 = acc0 + ek * w0
            acc1 = acc1 + ek * w1
        t_v[l, pl.ds(0, 16)] = acc0
        t_v[l, pl.ds(16, 16)] = acc1
    pltpu.sync_copy(t_v, t_hbm.at[pl.ds(wid * _ROWS_PER_W, _ROWS_PER_W)])


@functools.cache
def _make_gather(n_rows):
    rows_per_w = n_rows // _NW
    steps = rows_per_w // _K

    @functools.partial(
        pl.kernel,
        mesh=_MESH,
        compiler_params=_SC_PARAMS,
        out_type=jax.ShapeDtypeStruct((n_rows, _C, _DIM), jnp.float32),
        scratch_types=[
            pltpu.VMEM((2, _K, _C), jnp.int32),
            pltpu.VMEM((2, _K, _C, _DIM), jnp.float32),
            pltpu.SemaphoreType.DMA,
            pltpu.SemaphoreType.DMA,
            pltpu.SemaphoreType.DMA,
        ],
    )
    def sc_gather(idx_hbm, t_hbm, out_hbm, idx_v, out_v, sem_i, sem_g, sem_o):
        wid = lax.axis_index("s") * _NC + lax.axis_index("c")
        row0 = wid * rows_per_w

        def fire_idx(s, p):
            r = row0 + s * _K
            pltpu.async_copy(idx_hbm.at[pl.ds(r, _K)], idx_v.at[p], sem_i)

        def wait_idx(p):
            pltpu.make_async_copy(
                idx_hbm.at[pl.ds(row0, _K)], idx_v.at[p], sem_i).wait()

        def wait_out():
            pltpu.make_async_copy(
                out_v.at[0], out_hbm.at[pl.ds(row0, _K)], sem_o).wait()

        fire_idx(0, 0)

        def outer(o, carry):
            for p in range(2):
                s = o * 2 + p
                wait_idx(p)

                @pl.when(s + 1 < steps)
                def _prefetch():
                    fire_idx(s + 1, 1 - p)

                @pl.when(s >= 2)
                def _reclaim():
                    wait_out()

                copies = [
                    pltpu.async_copy(
                        t_hbm.at[idx_v.at[p].at[j]], out_v.at[p].at[j], sem_g)
                    for j in range(_K)
                ]
                for cp in copies:
                    cp.wait()
                pltpu.async_copy(
                    out_v.at[p], out_hbm.at[pl.ds(row0 + s * _K, _K)], sem_o)
            return carry

        lax.fori_loop(0, steps // 2, outer, 0)
        wait_out()
        wait_out()

    return sc_gather


def kernel(x, embed_table, W, b):
    bsz, hist = x.shape
    n = bsz * hist
    idx = x.reshape(n // _C, _C).astype(jnp.int32)
    e2 = jnp.pad(embed_table.astype(jnp.float32),
                 ((0, _VPAD - embed_table.shape[0]), (0, 0))).reshape(-1, 128)
    w2 = W.astype(jnp.float32).T.reshape(-1, 128)
    b2 = jnp.pad(b.astype(jnp.float32), (0, 128 - _DIM))
    table = _build_table(e2, w2, b2)
    out = _make_gather(n // _C)(idx, table)
    return out.reshape(bsz, hist, _DIM)


# speedup vs baseline: 1.6217x; 1.6217x over previous
import functools

import jax
import jax.numpy as jnp
from jax import lax
from jax.experimental import pallas as pl
from jax.experimental.pallas import tpu as pltpu
from jax.experimental.pallas import tpu_sc as plsc

_DIM = 32
_VPAD = 128
_C = 128
_K = 8
_NC = 2
_NS = 16
_NW = _NC * _NS
_BROWS = _VPAD // _NS


def _vbroadcast(vec, idx16):
    return lax.gather(
        vec, idx16[:, None],
        lax.GatherDimensionNumbers(
            offset_dims=(), collapsed_slice_dims=(0,), start_index_map=(0,)),
        (1,), mode=lax.GatherScatterMode.PROMISE_IN_BOUNDS)


@functools.cache
def _make_sc_kernel(n_rows):
    rows_per_w = n_rows // _NW
    steps = rows_per_w // _K
    mesh = plsc.VectorSubcoreMesh(core_axis_name="c", subcore_axis_name="s")

    @functools.partial(
        pl.kernel,
        mesh=mesh,
        compiler_params=pltpu.CompilerParams(
            needs_layout_passes=False, use_tc_tiling_on_sc=False),
        out_type=jax.ShapeDtypeStruct((n_rows, _C, _DIM), jnp.float32),
        scratch_types=[
            pltpu.VMEM((_BROWS // 4, 128), jnp.float32),
            pltpu.VMEM((_DIM * _DIM // 128, 128), jnp.float32),
            pltpu.VMEM((128,), jnp.float32),
            pltpu.VMEM((_BROWS, _DIM), jnp.float32),
            pltpu.VMEM_SHARED((_VPAD, _DIM), jnp.float32),
            pltpu.VMEM((_VPAD, _DIM), jnp.float32),
            pltpu.VMEM((2, _K, _C), jnp.int32),
            pltpu.VMEM((2, _K, _C, _DIM), jnp.float32),
            pltpu.SemaphoreType.DMA,
            pltpu.SemaphoreType.DMA,
        ],
    )
    def sc_kernel(idx_hbm, e_hbm, w_hbm, b_hbm, out_hbm,
                  e_vs, w_v, b_v, t_b, t_s, t2d, idx_v, out_v, sem_i, sem_o):
        sid = lax.axis_index("s")
        pltpu.sync_copy(e_hbm.at[pl.ds(sid * (_BROWS // 4), _BROWS // 4)], e_vs)
        pltpu.sync_copy(w_hbm, w_v)
        pltpu.sync_copy(b_hbm, b_v)
        b0 = b_v[pl.ds(0, 16)]
        b1 = b_v[pl.ds(16, 16)]

        for l in range(_BROWS):
            acc0, acc1 = b0, b1
            for k in range(_DIM):
                f = l * _DIM + k
                ek = plsc.load_gather(
                    e_vs, [jnp.full((16,), f // 128, dtype=jnp.int32),
                           jnp.full((16,), f % 128, dtype=jnp.int32)])
                wf = k * _DIM
                w0 = w_v[wf // 128, pl.ds(wf % 128, 16)]
                w1 = w_v[wf // 128, pl.ds(wf % 128 + 16, 16)]
                acc0 = acc0 + ek * w0
                acc1 = acc1 + ek * w1
            t_b[l, pl.ds(0, 16)] = acc0
            t_b[l, pl.ds(16, 16)] = acc1

        pltpu.sync_copy(t_b, t_s.at[pl.ds(sid * _BROWS, _BROWS)])
        plsc.subcore_barrier()
        pltpu.sync_copy(t_s, t2d)

        wid = sid * _NC + lax.axis_index("c")
        row0 = wid * rows_per_w

        def fire_idx(s, p):
            r = row0 + s * _K
            pltpu.async_copy(idx_hbm.at[pl.ds(r, _K)], idx_v.at[p], sem_i)

        def wait_idx(p):
            pltpu.make_async_copy(
                idx_hbm.at[pl.ds(row0, _K)], idx_v.at[p], sem_i).wait()

        def wait_out():
            pltpu.make_async_copy(
                out_v.at[0], out_hbm.at[pl.ds(row0, _K)], sem_o).wait()

        fire_idx(0, 0)
        cols0 = lax.iota(jnp.int32, 16)
        lsel = [jnp.full((16,), l, dtype=jnp.int32) for l in range(16)]

        def outer(o, carry):
            for p in range(2):
                s = o * 2 + p
                wait_idx(p)

                @pl.when(s + 1 < steps)
                def _prefetch():
                    fire_idx(s + 1, 1 - p)

                @pl.when(s >= 2)
                def _reclaim():
                    wait_out()

                ob = out_v.at[p]

                def grp(i, c2):
                    iv = idx_v[p, i >> 3, pl.ds((i & 7) * 16, 16)]
                    d0 = i >> 3
                    d1 = (i & 7) * 16
                    for l in range(16):
                        bvl = _vbroadcast(iv, lsel[l])
                        g0 = plsc.load_gather(t2d, [bvl, cols0])
                        g1 = plsc.load_gather(t2d, [bvl, cols0 + 16])
                        ob[d0, d1 + l, pl.ds(0, 16)] = g0
                        ob[d0, d1 + l, pl.ds(16, 16)] = g1
                    return c2

                lax.fori_loop(0, _K * (_C // 16), grp, 0)
                pltpu.async_copy(
                    ob, out_hbm.at[pl.ds(row0 + s * _K, _K)], sem_o)
            return carry

        lax.fori_loop(0, steps // 2, outer, 0)
        wait_out()
        wait_out()

    return sc_kernel


def kernel(x, embed_table, W, b):
    bsz, hist = x.shape
    n = bsz * hist
    idx = x.reshape(n // _C, _C).astype(jnp.int32)
    e2 = jnp.pad(embed_table.astype(jnp.float32),
                 ((0, _VPAD - embed_table.shape[0]), (0, 0))).reshape(-1, 128)
    w2 = W.astype(jnp.float32).T.reshape(-1, 128)
    b2 = jnp.pad(b.astype(jnp.float32), (0, 128 - _DIM))
    out = _make_sc_kernel(n // _C)(idx, e2, w2, b2)
    return out.reshape(bsz, hist, _DIM)
